# baseline (speedup 1.0000x reference)
import jax
import jax.numpy as jnp
from jax.experimental import pallas as pl
from jax.experimental.pallas import tpu as pltpu

N, D, DE, DT = 10000, 128, 32, 10
DIN = D + DE
BLOCK = 3336


def _body(x_ref, wz_ref, bz_ref, wh_ref, bh_ref, blin_ref, wp_ref, bp_ref,
          out_ref, h_ref):
    x = x_ref[...]
    w = jnp.concatenate(
        [wz_ref[0, 0, :D, :] + wz_ref[1, 0, :D, :],
         wh_ref[0, 0, :D, :] + wh_ref[1, 0, :D, :]], axis=1)
    g = jnp.dot(x, w, preferred_element_type=jnp.float32)
    z = jax.nn.sigmoid(g[:, :DE] + bz_ref[...])
    ht = jnp.tanh(g[:, DE:] + bh_ref[...])
    h = jnp.maximum((1.0 - z) * ht + blin_ref[...], 0.0)
    h_ref[...] = h
    out_ref[...] = (
        jnp.dot(h, wp_ref[...], preferred_element_type=jnp.float32)
        + bp_ref[...])


@jax.jit
def kernel(x, edge_index, mask, W_z, b_z, W_r, b_r, W_h, b_h,
           W_lin, b_lin, W_pred, b_pred):
    del edge_index, mask, W_r, b_r, W_lin
    grid = (pl.cdiv(N, BLOCK),)
    out, h = pl.pallas_call(
        _body,
        grid=grid,
        in_specs=[
            pl.BlockSpec((BLOCK, D), lambda i: (i, 0)),
            pl.BlockSpec((2, 1, DIN, DE), lambda i: (0, 0, 0, 0)),
            pl.BlockSpec((DE,), lambda i: (0,)),
            pl.BlockSpec((2, 1, DIN, DE), lambda i: (0, 0, 0, 0)),
            pl.BlockSpec((DE,), lambda i: (0,)),
            pl.BlockSpec((DE,), lambda i: (0,)),
            pl.BlockSpec((DE, DT), lambda i: (0, 0)),
            pl.BlockSpec((DT,), lambda i: (0,)),
        ],
        out_specs=[
            pl.BlockSpec((BLOCK, DT), lambda i: (i, 0)),
            pl.BlockSpec((BLOCK, DE), lambda i: (i, 0)),
        ],
        out_shape=[
            jax.ShapeDtypeStruct((N, DT), jnp.float32),
            jax.ShapeDtypeStruct((N, DE), jnp.float32),
        ],
        compiler_params=pltpu.CompilerParams(
            dimension_semantics=("parallel",)),
    )(x, W_z, b_z, W_h, b_h, b_lin, W_pred, b_pred)
    return (out, h)


# speedup vs baseline: 1.0540x; 1.0540x over previous
import jax
import jax.numpy as jnp
from jax.experimental import pallas as pl
from jax.experimental.pallas import tpu as pltpu

N, D, DE, DT = 10000, 128, 32, 10
DIN = D + DE
BLOCK = 5000


def _body(x_ref, wz_ref, bz_ref, wh_ref, bh_ref, blin_ref, wp_ref, bp_ref,
          out_ref, h_ref):
    x = x_ref[...]
    w = jnp.concatenate(
        [wz_ref[0, 0, :D, :] + wz_ref[1, 0, :D, :],
         wh_ref[0, 0, :D, :] + wh_ref[1, 0, :D, :]], axis=1)
    g = jnp.dot(x, w, preferred_element_type=jnp.float32)
    z = jax.nn.sigmoid(g[:, :DE] + bz_ref[...])
    ht = jnp.tanh(g[:, DE:] + bh_ref[...])
    h = jnp.maximum((1.0 - z) * ht + blin_ref[...], 0.0)
    h_ref[...] = h
    out_ref[...] = (
        jnp.dot(h, wp_ref[...], preferred_element_type=jnp.float32)
        + bp_ref[...])


@jax.jit
def kernel(x, edge_index, mask, W_z, b_z, W_r, b_r, W_h, b_h,
           W_lin, b_lin, W_pred, b_pred):
    del edge_index, mask, W_r, b_r, W_lin
    grid = (pl.cdiv(N, BLOCK),)
    out, h = pl.pallas_call(
        _body,
        grid=grid,
        in_specs=[
            pl.BlockSpec((BLOCK, D), lambda i: (i, 0)),
            pl.BlockSpec((2, 1, DIN, DE), lambda i: (0, 0, 0, 0)),
            pl.BlockSpec((DE,), lambda i: (0,)),
            pl.BlockSpec((2, 1, DIN, DE), lambda i: (0, 0, 0, 0)),
            pl.BlockSpec((DE,), lambda i: (0,)),
            pl.BlockSpec((DE,), lambda i: (0,)),
            pl.BlockSpec((DE, DT), lambda i: (0, 0)),
            pl.BlockSpec((DT,), lambda i: (0,)),
        ],
        out_specs=[
            pl.BlockSpec((BLOCK, DT), lambda i: (i, 0)),
            pl.BlockSpec((BLOCK, DE), lambda i: (i, 0)),
        ],
        out_shape=[
            jax.ShapeDtypeStruct((N, DT), jnp.float32),
            jax.ShapeDtypeStruct((N, DE), jnp.float32),
        ],
        compiler_params=pltpu.CompilerParams(
            dimension_semantics=("parallel",)),
    )(x, W_z, b_z, W_h, b_h, b_lin, W_pred, b_pred)
    return (out, h)
